# baseline (speedup 1.0000x reference)
import functools

import jax
import jax.numpy as jnp
from jax import lax
from jax.experimental import pallas as pl
from jax.experimental.pallas import tpu as pltpu
from jax.experimental.pallas import tpu_sc as plsc

_F32 = jnp.float32
_G = 2048
_GPAD = 2176
_B = 2000


def _make_edge_call(N, EPAD):
    CPT = EPAD // (16 * 128)
    IB = 32
    NBLK = CPT // IB
    NPAD = N + 48
    NZCH = NPAD // 128
    NZPT = (NZCH + 15) // 16
    RPT = NPAD // 16
    mesh = plsc.VectorSubcoreMesh(core_axis_name="c", subcore_axis_name="s")

    K = 2
    NGRP = IB // K

    @functools.partial(
        pl.kernel,
        out_type=jax.ShapeDtypeStruct((2, NPAD, 32), _F32),
        mesh=mesh,
        compiler_params=pltpu.CompilerParams(use_tc_tiling_on_sc=False),
        scratch_types=(
            pltpu.VMEM((2, K, 128, 32), _F32),
            pltpu.VMEM((IB, 128), jnp.int32),
            pltpu.VMEM((IB, 128), jnp.int32),
            pltpu.VMEM_SHARED((NPAD, 32), _F32),
            pltpu.SemaphoreType.DMA,
            pltpu.SemaphoreType.DMA,
        ),
    )
    def edge_call(eidx_hbm, dst_hbm, table_hbm, zeros_hbm, out_hbm,
                  rows_v, gidx_v, didx_v, acc_sh, sem_a, sem_b):
        c = lax.axis_index("c")
        s = lax.axis_index("s")

        def fire(g, p, sem):
            for k in range(K):
                pltpu.async_copy(table_hbm.at[gidx_v.at[g * K + k]],
                                 rows_v.at[p, k], sem)

        def drain(p, sem):
            for k in range(K):
                pltpu.make_async_copy(table_hbm.at[pl.ds(0, 128)],
                                      rows_v.at[p, k], sem).wait()

        def scatter(g, p):
            for k in range(K):
                pltpu.sync_copy(rows_v.at[p, k],
                                acc_sh.at[didx_v.at[g * K + k]], add=True)

        def za(z, _):
            rc = s + z * 16

            @pl.when(rc < NZCH)
            def _():
                pltpu.sync_copy(zeros_hbm, acc_sh.at[pl.ds(rc * 128, 128)])
            return 0
        lax.fori_loop(0, NZPT, za, 0)
        plsc.subcore_barrier()

        def blk(b, _):
            base = s * CPT + b * IB
            pltpu.sync_copy(eidx_hbm.at[c, pl.ds(base, IB)], gidx_v)
            pltpu.sync_copy(dst_hbm.at[pl.ds(base, IB)], didx_v)
            fire(0, 0, sem_a)

            def pair(hh, _):
                ga = 2 * hh
                gb = ga + 1

                @pl.when(gb < NGRP)
                def _():
                    fire(gb, 1, sem_b)
                drain(0, sem_a)
                scatter(ga, 0)

                @pl.when(ga + 2 < NGRP)
                def _():
                    fire(ga + 2, 0, sem_a)

                @pl.when(gb < NGRP)
                def _():
                    drain(1, sem_b)
                    scatter(gb, 1)
                return 0
            lax.fori_loop(0, (NGRP + 1) // 2, pair, 0)
            return 0
        lax.fori_loop(0, NBLK, blk, 0)
        plsc.subcore_barrier()

        pltpu.sync_copy(acc_sh.at[pl.ds(s * RPT, RPT)],
                        out_hbm.at[c, pl.ds(s * RPT, RPT)])

    return edge_call


def _make_pool_call(N, NPCH):
    CPTP = NPCH // 32
    ZPT = _GPAD // 16
    mesh = plsc.VectorSubcoreMesh(core_axis_name="c", subcore_axis_name="s")

    @functools.partial(
        pl.kernel,
        out_type=(jax.ShapeDtypeStruct((2, _GPAD, 64), _F32),
                  jax.ShapeDtypeStruct((2, _GPAD, 16), _F32)),
        mesh=mesh,
        compiler_params=pltpu.CompilerParams(use_tc_tiling_on_sc=False),
        scratch_types=(
            pltpu.VMEM((128, 64), _F32),
            pltpu.VMEM((128, 16), _F32),
            pltpu.VMEM((CPTP, 128), jnp.int32),
            pltpu.VMEM_SHARED((_GPAD, 64), _F32),
            pltpu.VMEM_SHARED((_GPAD, 16), _F32),
        ),
    )
    def pool_call(h_hbm, bidx_hbm, zeros_hbm, zerosb_hbm, ones_hbm,
                  outa_hbm, outb_hbm, rows_v, ones_v, bidx_v,
                  acca_sh, accb_sh):
        c = lax.axis_index("c")
        s = lax.axis_index("s")
        w = c * 16 + s
        pltpu.sync_copy(zeros_hbm, acca_sh.at[pl.ds(s * ZPT, ZPT)])
        pltpu.sync_copy(zerosb_hbm, accb_sh.at[pl.ds(s * ZPT, ZPT)])
        pltpu.sync_copy(ones_hbm, ones_v)
        pltpu.sync_copy(bidx_hbm.at[pl.ds(w * CPTP, CPTP)], bidx_v)
        plsc.subcore_barrier()

        def ch(j, _):
            chunk = w * CPTP + j
            rbase = jnp.minimum(chunk * 128, N - 128)
            pltpu.sync_copy(h_hbm.at[pl.ds(rbase, 128)], rows_v)
            pltpu.sync_copy(rows_v, acca_sh.at[bidx_v.at[j]], add=True)
            pltpu.sync_copy(ones_v, accb_sh.at[bidx_v.at[j]], add=True)
            return 0
        lax.fori_loop(0, CPTP, ch, 0)
        plsc.subcore_barrier()

        pltpu.sync_copy(acca_sh.at[pl.ds(s * ZPT, ZPT)],
                        outa_hbm.at[c, pl.ds(s * ZPT, ZPT)])
        pltpu.sync_copy(accb_sh.at[pl.ds(s * ZPT, ZPT)],
                        outb_hbm.at[c, pl.ds(s * ZPT, ZPT)])

    return pool_call


def _row_spec(i_map=None):
    return pl.BlockSpec((_B, 64), i_map or (lambda i: (i, 0)))


_FULL64 = pl.BlockSpec((64, 64), lambda i: (0, 0))
_VEC = pl.BlockSpec((1, 64), lambda i: (0, 0))
_ST = pl.BlockSpec((8, 64), lambda i: (0, 0))


def _accum_stats(st_ref, y, first):
    @pl.when(first)
    def _():
        st_ref[...] = jnp.zeros_like(st_ref)
        st_ref[2:3, :] = jnp.mean(y, axis=0, keepdims=True)
    d = y - st_ref[2:3, :]
    st_ref[0:1, :] += jnp.sum(d, axis=0, keepdims=True)
    st_ref[1:2, :] += jnp.sum(d * d, axis=0, keepdims=True)


def _bn_coeffs(st, n, g, b):
    dm = st[0:1, :] / n
    m = st[2:3, :] + dm
    v = st[1:2, :] / n - dm * dm
    a = g / jnp.sqrt(v + 1e-5)
    return a, b - m * a


def _write_r(r_ref, h, bond):
    rb = jax.nn.relu(h[None, :, :] + bond[:, None, :])
    r_ref[0] = rb[:, :, 0:32]
    r_ref[1] = rb[:, :, 32:64]


def _pre_a_body(x_ref, pe_ref, atom_ref, pw1_ref, pb1_ref, pw2_ref, pb2_ref,
                inw_ref, inb_ref, p1_ref, hp_ref, st_ref):
    i = pl.program_id(0)
    xv = x_ref[0, 0, :]
    oh = (xv[:, None] == lax.broadcasted_iota(jnp.int32, (_B, 32), 1)
          ).astype(_F32)
    h0 = jnp.dot(oh, atom_ref[...], preferred_element_type=_F32)
    hp_ref[...] = jnp.dot(h0, inw_ref[0:64, :],
                          preferred_element_type=_F32) + inb_ref[...]
    u = jax.nn.relu(jnp.dot(pe_ref[...], pw1_ref[...],
                            preferred_element_type=_F32) + pb1_ref[...])
    p1 = jnp.dot(u, pw2_ref[...], preferred_element_type=_F32) + pb2_ref[...]
    p1_ref[...] = p1
    _accum_stats(st_ref, p1, i == 0)


def _pre_b_body(n, p1_ref, hp_ref, st_ref, g_ref, b_ref, inw_ref, bond_ref,
                h_ref, r_ref):
    a, cc = _bn_coeffs(st_ref[...], n, g_ref[...], b_ref[...])
    p = p1_ref[...] * a + cc
    h = hp_ref[...] + jnp.dot(p, inw_ref[64:128, :],
                              preferred_element_type=_F32)
    h_ref[...] = h
    _write_r(r_ref, h, bond_ref[...])


def _p1_body(h_ref, agg_ref, w_ref, b_ref, y_ref, st_ref):
    i = pl.program_id(0)
    z = h_ref[...] + jnp.concatenate([agg_ref[0], agg_ref[1]], axis=1)
    y = jnp.dot(z, w_ref[...], preferred_element_type=_F32) + b_ref[...]
    y_ref[...] = y
    _accum_stats(st_ref, y, i == 0)


def _p2_body(n, y1_ref, st1_ref, g_ref, b_ref, w_ref, bb_ref, y_ref, st_ref):
    i = pl.program_id(0)
    a, cc = _bn_coeffs(st1_ref[...], n, g_ref[...], b_ref[...])
    x2 = jax.nn.relu(y1_ref[...] * a + cc)
    y = jnp.dot(x2, w_ref[...], preferred_element_type=_F32) + bb_ref[...]
    y_ref[...] = y
    _accum_stats(st_ref, y, i == 0)


def _p3_body(n, y2_ref, st_ref, g_ref, b_ref, bond_ref, h_ref, r_ref):
    a, cc = _bn_coeffs(st_ref[...], n, g_ref[...], b_ref[...])
    h = jax.nn.relu(y2_ref[...] * a + cc)
    h_ref[...] = h
    _write_r(r_ref, h, bond_ref[...])


def _p3_last_body(n, y2_ref, st_ref, g_ref, b_ref, h_ref):
    a, cc = _bn_coeffs(st_ref[...], n, g_ref[...], b_ref[...])
    h_ref[...] = jax.nn.relu(y2_ref[...] * a + cc)


def _ro_body(n, acca_ref, accb_ref, w1_ref, b1_ref, w2_ref, b2_ref, o_ref):
    xa = acca_ref[0] + acca_ref[1]
    cnt = accb_ref[0][:, 0:1] + accb_ref[1][:, 0:1]
    mean = xa / jnp.maximum(cnt, 1.0)
    r = jnp.concatenate([xa, mean], axis=1)
    rh = jax.nn.relu(jnp.dot(r, w1_ref[...], preferred_element_type=_F32)
                     + b1_ref[...])
    o_ref[...] = jnp.sum(rh * w2_ref[...], axis=1, keepdims=True) + b2_ref[...]


def kernel(x, pe, edge_index, edge_attr, batch, atom_emb, bond_emb, pe_w1, pe_b1, pe_w2, pe_b2, pe_bn_g, pe_bn_b, in_w, in_b, conv_w1, conv_b1, conv_bn1_g, conv_bn1_b, conv_w2, conv_b2, bn_g, bn_b, ro_w1, ro_b1, ro_w2, ro_b2):
    N = x.shape[0]
    E = edge_index.shape[1]
    L = conv_w1.shape[0]
    T = N // _B
    nf = float(N)

    CPT = -(-E // (16 * 128))
    CPT = ((CPT + 31) // 32) * 32
    EPAD = CPT * 16 * 128
    src = edge_index[0].astype(jnp.int32)
    dst = edge_index[1].astype(jnp.int32)
    gidx = edge_attr.astype(jnp.int32) * N + src
    gidx_p = jnp.concatenate([gidx, jnp.zeros((EPAD - E,), jnp.int32)])
    eidx2 = jnp.stack([gidx_p, gidx_p + 4 * N]).reshape(2, EPAD // 128, 128)
    dst_p = jnp.concatenate(
        [dst, jnp.full((EPAD - E,), N, jnp.int32)]).reshape(EPAD // 128, 128)
    zeros_blk = jnp.zeros((128, 32), _F32)

    NPCH = 512
    batch32 = batch.astype(jnp.int32)
    nfull = N // 128
    rem = N - nfull * 128
    rows = [batch32[:nfull * 128].reshape(nfull, 128)]
    used = nfull
    if rem:
        kk = jnp.arange(128, dtype=jnp.int32)
        rows.append(jnp.where(kk < 128 - rem, _G, batch32[N - 128:])[None])
        used += 1
    rows.append(jnp.full((NPCH - used, 128), _G, jnp.int32))
    bidx = jnp.concatenate(rows, axis=0)
    zeros_pool = jnp.zeros((_GPAD // 16, 64), _F32)
    zerosb_pool = jnp.zeros((_GPAD // 16, 16), _F32)
    ones_pool = jnp.ones((128, 16), _F32)

    atom32 = jnp.concatenate([atom_emb, jnp.zeros((4, 64), _F32)], axis=0)
    x3 = x.astype(jnp.int32).reshape(T, 1, _B)

    edge_call = _make_edge_call(N, EPAD)
    pool_call = _make_pool_call(N, NPCH)

    p1, hp, st0 = pl.pallas_call(
        _pre_a_body,
        grid=(T,),
        in_specs=[
            pl.BlockSpec((1, 1, _B), lambda i: (i, 0, 0)),
            pl.BlockSpec((_B, 16), lambda i: (i, 0)),
            pl.BlockSpec((32, 64), lambda i: (0, 0)),
            pl.BlockSpec((16, 64), lambda i: (0, 0)),
            _VEC, _FULL64, _VEC,
            pl.BlockSpec((128, 64), lambda i: (0, 0)),
            _VEC,
        ],
        out_specs=[_row_spec(), _row_spec(), _ST],
        out_shape=[jax.ShapeDtypeStruct((N, 64), _F32),
                   jax.ShapeDtypeStruct((N, 64), _F32),
                   jax.ShapeDtypeStruct((8, 64), _F32)],
    )(x3, pe, atom32, pe_w1, pe_b1[None], pe_w2, pe_b2[None],
      in_w, in_b[None])

    h, R = pl.pallas_call(
        functools.partial(_pre_b_body, nf),
        grid=(T,),
        in_specs=[
            _row_spec(), _row_spec(), _ST, _VEC, _VEC,
            pl.BlockSpec((128, 64), lambda i: (0, 0)),
            pl.BlockSpec((4, 64), lambda i: (0, 0)),
        ],
        out_specs=[_row_spec(),
                   pl.BlockSpec((2, 4, _B, 32), lambda i: (0, 0, i, 0))],
        out_shape=[jax.ShapeDtypeStruct((N, 64), _F32),
                   jax.ShapeDtypeStruct((2, 4, N, 32), _F32)],
    )(p1, hp, st0, pe_bn_g[None], pe_bn_b[None], in_w, bond_emb)

    for l in range(L):
        table = R.reshape(8 * N, 32)
        agg2 = edge_call(eidx2, dst_p, table, zeros_blk)
        y1, st1 = pl.pallas_call(
            _p1_body,
            grid=(T,),
            in_specs=[
                _row_spec(),
                pl.BlockSpec((2, _B, 32), lambda i: (0, i, 0)),
                _FULL64, _VEC,
            ],
            out_specs=[_row_spec(), _ST],
            out_shape=[jax.ShapeDtypeStruct((N, 64), _F32),
                       jax.ShapeDtypeStruct((8, 64), _F32)],
        )(h, agg2, conv_w1[l], conv_b1[l][None])
        y2, st2 = pl.pallas_call(
            functools.partial(_p2_body, nf),
            grid=(T,),
            in_specs=[_row_spec(), _ST, _VEC, _VEC, _FULL64, _VEC],
            out_specs=[_row_spec(), _ST],
            out_shape=[jax.ShapeDtypeStruct((N, 64), _F32),
                       jax.ShapeDtypeStruct((8, 64), _F32)],
        )(y1, st1, conv_bn1_g[l][None], conv_bn1_b[l][None],
          conv_w2[l], conv_b2[l][None])
        if l + 1 < L:
            h, R = pl.pallas_call(
                functools.partial(_p3_body, nf),
                grid=(T,),
                in_specs=[_row_spec(), _ST, _VEC, _VEC,
                          pl.BlockSpec((4, 64), lambda i: (0, 0))],
                out_specs=[_row_spec(),
                           pl.BlockSpec((2, 4, _B, 32),
                                        lambda i: (0, 0, i, 0))],
                out_shape=[jax.ShapeDtypeStruct((N, 64), _F32),
                           jax.ShapeDtypeStruct((2, 4, N, 32), _F32)],
            )(y2, st2, bn_g[l][None], bn_b[l][None], bond_emb)
        else:
            h = pl.pallas_call(
                functools.partial(_p3_last_body, nf),
                grid=(T,),
                in_specs=[_row_spec(), _ST, _VEC, _VEC],
                out_specs=_row_spec(),
                out_shape=jax.ShapeDtypeStruct((N, 64), _F32),
            )(y2, st2, bn_g[l][None], bn_b[l][None])

    acca, accb = pool_call(h, bidx, zeros_pool, zerosb_pool, ones_pool)
    out = pl.pallas_call(
        functools.partial(_ro_body, nf),
        grid=(1,),
        in_specs=[
            pl.BlockSpec((2, _G, 64), lambda i: (0, 0, 0)),
            pl.BlockSpec((2, _G, 16), lambda i: (0, 0, 0)),
            pl.BlockSpec((128, 64), lambda i: (0, 0)),
            pl.BlockSpec((1, 64), lambda i: (0, 0)),
            pl.BlockSpec((1, 64), lambda i: (0, 0)),
            pl.BlockSpec((1, 1), lambda i: (0, 0)),
        ],
        out_specs=pl.BlockSpec((_G, 1), lambda i: (0, 0)),
        out_shape=jax.ShapeDtypeStruct((_G, 1), _F32),
    )(acca, accb, ro_w1, ro_b1[None], ro_w2.T, ro_b2[None])
    return out[:, 0]


# speedup vs baseline: 1.1858x; 1.1858x over previous
import functools

import jax
import jax.numpy as jnp
from jax import lax
from jax.experimental import pallas as pl
from jax.experimental.pallas import tpu as pltpu
from jax.experimental.pallas import tpu_sc as plsc

_F32 = jnp.float32
_G = 2048
_GPAD = 2176
_B = 2000


def _make_edge_call(N, EPAD):
    CPT = EPAD // (16 * 128)
    IB = 24
    NBLK = CPT // IB
    NPAD = N + 48
    NZCH = NPAD // 128
    NZPT = (NZCH + 15) // 16
    RPT = NPAD // 16
    mesh = plsc.VectorSubcoreMesh(core_axis_name="c", subcore_axis_name="s")

    K = 3
    NGRP = IB // K

    @functools.partial(
        pl.kernel,
        out_type=jax.ShapeDtypeStruct((2, NPAD, 32), _F32),
        mesh=mesh,
        compiler_params=pltpu.CompilerParams(use_tc_tiling_on_sc=False),
        scratch_types=(
            pltpu.VMEM((2, K, 128, 32), _F32),
            pltpu.VMEM((IB, 128), jnp.int32),
            pltpu.VMEM((IB, 128), jnp.int32),
            pltpu.VMEM_SHARED((NPAD, 32), _F32),
            pltpu.SemaphoreType.DMA,
            pltpu.SemaphoreType.DMA,
        ),
    )
    def edge_call(eidx_hbm, dst_hbm, table_hbm, zeros_hbm, out_hbm,
                  rows_v, gidx_v, didx_v, acc_sh, sem_a, sem_b):
        c = lax.axis_index("c")
        s = lax.axis_index("s")

        def fire(g, p, sem):
            for k in range(K):
                pltpu.async_copy(table_hbm.at[gidx_v.at[g * K + k]],
                                 rows_v.at[p, k], sem)

        def drain(p, sem):
            for k in range(K):
                pltpu.make_async_copy(table_hbm.at[pl.ds(0, 128)],
                                      rows_v.at[p, k], sem).wait()

        def scatter(g, p):
            for k in range(K):
                pltpu.sync_copy(rows_v.at[p, k],
                                acc_sh.at[didx_v.at[g * K + k]], add=True)

        def za(z, _):
            rc = s + z * 16

            @pl.when(rc < NZCH)
            def _():
                pltpu.sync_copy(zeros_hbm, acc_sh.at[pl.ds(rc * 128, 128)])
            return 0
        lax.fori_loop(0, NZPT, za, 0)
        plsc.subcore_barrier()

        def blk(b, _):
            base = s * CPT + b * IB
            pltpu.sync_copy(eidx_hbm.at[c, pl.ds(base, IB)], gidx_v)
            pltpu.sync_copy(dst_hbm.at[pl.ds(base, IB)], didx_v)
            fire(0, 0, sem_a)

            def pair(hh, _):
                ga = 2 * hh
                gb = ga + 1

                @pl.when(gb < NGRP)
                def _():
                    fire(gb, 1, sem_b)
                drain(0, sem_a)
                scatter(ga, 0)

                @pl.when(ga + 2 < NGRP)
                def _():
                    fire(ga + 2, 0, sem_a)

                @pl.when(gb < NGRP)
                def _():
                    drain(1, sem_b)
                    scatter(gb, 1)
                return 0
            lax.fori_loop(0, (NGRP + 1) // 2, pair, 0)
            return 0
        lax.fori_loop(0, NBLK, blk, 0)
        plsc.subcore_barrier()

        pltpu.sync_copy(acc_sh.at[pl.ds(s * RPT, RPT)],
                        out_hbm.at[c, pl.ds(s * RPT, RPT)])

    return edge_call


def _make_pool_call(N, NPCH):
    CPTP = NPCH // 32
    ZPT = _GPAD // 16
    mesh = plsc.VectorSubcoreMesh(core_axis_name="c", subcore_axis_name="s")

    @functools.partial(
        pl.kernel,
        out_type=(jax.ShapeDtypeStruct((2, _GPAD, 64), _F32),
                  jax.ShapeDtypeStruct((2, _GPAD, 16), _F32)),
        mesh=mesh,
        compiler_params=pltpu.CompilerParams(use_tc_tiling_on_sc=False),
        scratch_types=(
            pltpu.VMEM((128, 64), _F32),
            pltpu.VMEM((128, 16), _F32),
            pltpu.VMEM((CPTP, 128), jnp.int32),
            pltpu.VMEM_SHARED((_GPAD, 64), _F32),
            pltpu.VMEM_SHARED((_GPAD, 16), _F32),
        ),
    )
    def pool_call(h_hbm, bidx_hbm, zeros_hbm, zerosb_hbm, ones_hbm,
                  outa_hbm, outb_hbm, rows_v, ones_v, bidx_v,
                  acca_sh, accb_sh):
        c = lax.axis_index("c")
        s = lax.axis_index("s")
        w = c * 16 + s
        pltpu.sync_copy(zeros_hbm, acca_sh.at[pl.ds(s * ZPT, ZPT)])
        pltpu.sync_copy(zerosb_hbm, accb_sh.at[pl.ds(s * ZPT, ZPT)])
        pltpu.sync_copy(ones_hbm, ones_v)
        pltpu.sync_copy(bidx_hbm.at[pl.ds(w * CPTP, CPTP)], bidx_v)
        plsc.subcore_barrier()

        def ch(j, _):
            chunk = w * CPTP + j
            rbase = jnp.minimum(chunk * 128, N - 128)
            pltpu.sync_copy(h_hbm.at[pl.ds(rbase, 128)], rows_v)
            pltpu.sync_copy(rows_v, acca_sh.at[bidx_v.at[j]], add=True)
            pltpu.sync_copy(ones_v, accb_sh.at[bidx_v.at[j]], add=True)
            return 0
        lax.fori_loop(0, CPTP, ch, 0)
        plsc.subcore_barrier()

        pltpu.sync_copy(acca_sh.at[pl.ds(s * ZPT, ZPT)],
                        outa_hbm.at[c, pl.ds(s * ZPT, ZPT)])
        pltpu.sync_copy(accb_sh.at[pl.ds(s * ZPT, ZPT)],
                        outb_hbm.at[c, pl.ds(s * ZPT, ZPT)])

    return pool_call


def _row_spec(i_map=None):
    return pl.BlockSpec((_B, 64), i_map or (lambda i: (i, 0)))


_FULL64 = pl.BlockSpec((64, 64), lambda i: (0, 0))
_VEC = pl.BlockSpec((1, 64), lambda i: (0, 0))
_ST = pl.BlockSpec((8, 64), lambda i: (0, 0))


def _accum_stats(st_ref, y, first):
    @pl.when(first)
    def _():
        st_ref[...] = jnp.zeros_like(st_ref)
        st_ref[2:3, :] = jnp.mean(y, axis=0, keepdims=True)
    d = y - st_ref[2:3, :]
    st_ref[0:1, :] += jnp.sum(d, axis=0, keepdims=True)
    st_ref[1:2, :] += jnp.sum(d * d, axis=0, keepdims=True)


def _bn_coeffs(st, n, g, b):
    dm = st[0:1, :] / n
    m = st[2:3, :] + dm
    v = st[1:2, :] / n - dm * dm
    a = g / jnp.sqrt(v + 1e-5)
    return a, b - m * a


def _write_r(r_ref, h, bond):
    rb = jax.nn.relu(h[None, :, :] + bond[:, None, :])
    r_ref[0] = rb[:, :, 0:32]
    r_ref[1] = rb[:, :, 32:64]


def _pre_a_body(x_ref, pe_ref, atom_ref, pw1_ref, pb1_ref, pw2_ref, pb2_ref,
                inw_ref, inb_ref, p1_ref, hp_ref, st_ref):
    i = pl.program_id(0)
    xv = x_ref[0, 0, :]
    oh = (xv[:, None] == lax.broadcasted_iota(jnp.int32, (_B, 32), 1)
          ).astype(_F32)
    h0 = jnp.dot(oh, atom_ref[...], preferred_element_type=_F32)
    hp_ref[...] = jnp.dot(h0, inw_ref[0:64, :],
                          preferred_element_type=_F32) + inb_ref[...]
    u = jax.nn.relu(jnp.dot(pe_ref[...], pw1_ref[...],
                            preferred_element_type=_F32) + pb1_ref[...])
    p1 = jnp.dot(u, pw2_ref[...], preferred_element_type=_F32) + pb2_ref[...]
    p1_ref[...] = p1
    _accum_stats(st_ref, p1, i == 0)


def _pre_b_body(n, p1_ref, hp_ref, st_ref, g_ref, b_ref, inw_ref, bond_ref,
                h_ref, r_ref):
    a, cc = _bn_coeffs(st_ref[...], n, g_ref[...], b_ref[...])
    p = p1_ref[...] * a + cc
    h = hp_ref[...] + jnp.dot(p, inw_ref[64:128, :],
                              preferred_element_type=_F32)
    h_ref[...] = h
    _write_r(r_ref, h, bond_ref[...])


def _p1_body(h_ref, agg_ref, w_ref, b_ref, y_ref, st_ref):
    i = pl.program_id(0)
    z = h_ref[...] + jnp.concatenate([agg_ref[0], agg_ref[1]], axis=1)
    y = jnp.dot(z, w_ref[...], preferred_element_type=_F32) + b_ref[...]
    y_ref[...] = y
    _accum_stats(st_ref, y, i == 0)


def _p2_body(n, y1_ref, st1_ref, g_ref, b_ref, w_ref, bb_ref, y_ref, st_ref):
    i = pl.program_id(0)
    a, cc = _bn_coeffs(st1_ref[...], n, g_ref[...], b_ref[...])
    x2 = jax.nn.relu(y1_ref[...] * a + cc)
    y = jnp.dot(x2, w_ref[...], preferred_element_type=_F32) + bb_ref[...]
    y_ref[...] = y
    _accum_stats(st_ref, y, i == 0)


def _p3_body(n, y2_ref, st_ref, g_ref, b_ref, bond_ref, h_ref, r_ref):
    a, cc = _bn_coeffs(st_ref[...], n, g_ref[...], b_ref[...])
    h = jax.nn.relu(y2_ref[...] * a + cc)
    h_ref[...] = h
    _write_r(r_ref, h, bond_ref[...])


def _p3_last_body(n, y2_ref, st_ref, g_ref, b_ref, h_ref):
    a, cc = _bn_coeffs(st_ref[...], n, g_ref[...], b_ref[...])
    h_ref[...] = jax.nn.relu(y2_ref[...] * a + cc)


def _ro_body(n, acca_ref, accb_ref, w1_ref, b1_ref, w2_ref, b2_ref, o_ref):
    xa = acca_ref[0] + acca_ref[1]
    cnt = accb_ref[0][:, 0:1] + accb_ref[1][:, 0:1]
    mean = xa / jnp.maximum(cnt, 1.0)
    r = jnp.concatenate([xa, mean], axis=1)
    rh = jax.nn.relu(jnp.dot(r, w1_ref[...], preferred_element_type=_F32)
                     + b1_ref[...])
    o_ref[...] = jnp.sum(rh * w2_ref[...], axis=1, keepdims=True) + b2_ref[...]


def kernel(x, pe, edge_index, edge_attr, batch, atom_emb, bond_emb, pe_w1, pe_b1, pe_w2, pe_b2, pe_bn_g, pe_bn_b, in_w, in_b, conv_w1, conv_b1, conv_bn1_g, conv_bn1_b, conv_w2, conv_b2, bn_g, bn_b, ro_w1, ro_b1, ro_w2, ro_b2):
    N = x.shape[0]
    E = edge_index.shape[1]
    L = conv_w1.shape[0]
    T = N // _B
    nf = float(N)

    CPT = -(-E // (16 * 128))
    CPT = ((CPT + 23) // 24) * 24
    EPAD = CPT * 16 * 128
    src = edge_index[0].astype(jnp.int32)
    dst = edge_index[1].astype(jnp.int32)
    gidx = edge_attr.astype(jnp.int32) * N + src
    gidx_p = jnp.concatenate([gidx, jnp.zeros((EPAD - E,), jnp.int32)])
    eidx2 = jnp.stack([gidx_p, gidx_p + 4 * N]).reshape(2, EPAD // 128, 128)
    dst_p = jnp.concatenate(
        [dst, jnp.full((EPAD - E,), N, jnp.int32)]).reshape(EPAD // 128, 128)
    zeros_blk = jnp.zeros((128, 32), _F32)

    NPCH = 512
    batch32 = batch.astype(jnp.int32)
    nfull = N // 128
    rem = N - nfull * 128
    rows = [batch32[:nfull * 128].reshape(nfull, 128)]
    used = nfull
    if rem:
        kk = jnp.arange(128, dtype=jnp.int32)
        rows.append(jnp.where(kk < 128 - rem, _G, batch32[N - 128:])[None])
        used += 1
    rows.append(jnp.full((NPCH - used, 128), _G, jnp.int32))
    bidx = jnp.concatenate(rows, axis=0)
    zeros_pool = jnp.zeros((_GPAD // 16, 64), _F32)
    zerosb_pool = jnp.zeros((_GPAD // 16, 16), _F32)
    ones_pool = jnp.ones((128, 16), _F32)

    atom32 = jnp.concatenate([atom_emb, jnp.zeros((4, 64), _F32)], axis=0)
    x3 = x.astype(jnp.int32).reshape(T, 1, _B)

    edge_call = _make_edge_call(N, EPAD)
    pool_call = _make_pool_call(N, NPCH)

    p1, hp, st0 = pl.pallas_call(
        _pre_a_body,
        grid=(T,),
        in_specs=[
            pl.BlockSpec((1, 1, _B), lambda i: (i, 0, 0)),
            pl.BlockSpec((_B, 16), lambda i: (i, 0)),
            pl.BlockSpec((32, 64), lambda i: (0, 0)),
            pl.BlockSpec((16, 64), lambda i: (0, 0)),
            _VEC, _FULL64, _VEC,
            pl.BlockSpec((128, 64), lambda i: (0, 0)),
            _VEC,
        ],
        out_specs=[_row_spec(), _row_spec(), _ST],
        out_shape=[jax.ShapeDtypeStruct((N, 64), _F32),
                   jax.ShapeDtypeStruct((N, 64), _F32),
                   jax.ShapeDtypeStruct((8, 64), _F32)],
    )(x3, pe, atom32, pe_w1, pe_b1[None], pe_w2, pe_b2[None],
      in_w, in_b[None])

    h, R = pl.pallas_call(
        functools.partial(_pre_b_body, nf),
        grid=(T,),
        in_specs=[
            _row_spec(), _row_spec(), _ST, _VEC, _VEC,
            pl.BlockSpec((128, 64), lambda i: (0, 0)),
            pl.BlockSpec((4, 64), lambda i: (0, 0)),
        ],
        out_specs=[_row_spec(),
                   pl.BlockSpec((2, 4, _B, 32), lambda i: (0, 0, i, 0))],
        out_shape=[jax.ShapeDtypeStruct((N, 64), _F32),
                   jax.ShapeDtypeStruct((2, 4, N, 32), _F32)],
    )(p1, hp, st0, pe_bn_g[None], pe_bn_b[None], in_w, bond_emb)

    for l in range(L):
        table = R.reshape(8 * N, 32)
        agg2 = edge_call(eidx2, dst_p, table, zeros_blk)
        y1, st1 = pl.pallas_call(
            _p1_body,
            grid=(T,),
            in_specs=[
                _row_spec(),
                pl.BlockSpec((2, _B, 32), lambda i: (0, i, 0)),
                _FULL64, _VEC,
            ],
            out_specs=[_row_spec(), _ST],
            out_shape=[jax.ShapeDtypeStruct((N, 64), _F32),
                       jax.ShapeDtypeStruct((8, 64), _F32)],
        )(h, agg2, conv_w1[l], conv_b1[l][None])
        y2, st2 = pl.pallas_call(
            functools.partial(_p2_body, nf),
            grid=(T,),
            in_specs=[_row_spec(), _ST, _VEC, _VEC, _FULL64, _VEC],
            out_specs=[_row_spec(), _ST],
            out_shape=[jax.ShapeDtypeStruct((N, 64), _F32),
                       jax.ShapeDtypeStruct((8, 64), _F32)],
        )(y1, st1, conv_bn1_g[l][None], conv_bn1_b[l][None],
          conv_w2[l], conv_b2[l][None])
        if l + 1 < L:
            h, R = pl.pallas_call(
                functools.partial(_p3_body, nf),
                grid=(T,),
                in_specs=[_row_spec(), _ST, _VEC, _VEC,
                          pl.BlockSpec((4, 64), lambda i: (0, 0))],
                out_specs=[_row_spec(),
                           pl.BlockSpec((2, 4, _B, 32),
                                        lambda i: (0, 0, i, 0))],
                out_shape=[jax.ShapeDtypeStruct((N, 64), _F32),
                           jax.ShapeDtypeStruct((2, 4, N, 32), _F32)],
            )(y2, st2, bn_g[l][None], bn_b[l][None], bond_emb)
        else:
            h = pl.pallas_call(
                functools.partial(_p3_last_body, nf),
                grid=(T,),
                in_specs=[_row_spec(), _ST, _VEC, _VEC],
                out_specs=_row_spec(),
                out_shape=jax.ShapeDtypeStruct((N, 64), _F32),
            )(y2, st2, bn_g[l][None], bn_b[l][None])

    acca, accb = pool_call(h, bidx, zeros_pool, zerosb_pool, ones_pool)
    out = pl.pallas_call(
        functools.partial(_ro_body, nf),
        grid=(1,),
        in_specs=[
            pl.BlockSpec((2, _G, 64), lambda i: (0, 0, 0)),
            pl.BlockSpec((2, _G, 16), lambda i: (0, 0, 0)),
            pl.BlockSpec((128, 64), lambda i: (0, 0)),
            pl.BlockSpec((1, 64), lambda i: (0, 0)),
            pl.BlockSpec((1, 64), lambda i: (0, 0)),
            pl.BlockSpec((1, 1), lambda i: (0, 0)),
        ],
        out_specs=pl.BlockSpec((_G, 1), lambda i: (0, 0)),
        out_shape=jax.ShapeDtypeStruct((_G, 1), _F32),
    )(acca, accb, ro_w1, ro_b1[None], ro_w2.T, ro_b2[None])
    return out[:, 0]
